# Initial kernel scaffold; baseline (speedup 1.0000x reference)
import jax
import jax.numpy as jnp
from jax.experimental import pallas as pl


def kernel(x, feature1, feature2, edge_index, batch, lin_W, lin_b, f1_W1, f1_W2, f2_W1, f2_W2, c1_Wrel, c1_brel, c1_Wroot, c2_Wrel, c2_brel, c2_Wroot, lin1_W, lin1_b, lin2_W, lin2_b, cat_W, cat_b, lins_W, lins_b, gn_w, gn_b, gn_ms, final_W, final_b):
    raise NotImplementedError("write your pallas kernel here")


# speedup vs baseline: 1.3090x; 1.3090x over previous
import jax
import jax.numpy as jnp
from jax import lax
from jax.experimental import pallas as pl
from jax.experimental.pallas import tpu as pltpu
from jax.experimental.pallas import tpu_sc as plsc

N = 10000
NP = 10240
E = 320000
H = 128
MID = 64
F1 = 147
F2 = 21
NL = 4
NG = 64

NUM_CORES = 2
NUM_SUBCORES = 16
NW = NUM_CORES * NUM_SUBCORES
EW = E // NW
CHUNK = 80
NCHUNK = EW // CHUNK
ROWS = NP // NUM_SUBCORES
ZROWS = 128


def _dotT(a, w):
    return lax.dot_general(a, w, (((1,), (1,)), ((), ())),
                           preferred_element_type=jnp.float32)


def _swish(v):
    return v * jax.nn.sigmoid(v)


def _x1_body(x_ref, w_ref, b_ref, o_ref):
    o_ref[...] = _swish(_dotT(x_ref[...], w_ref[...]) + b_ref[...])


def _edge_body(feat1_ref, feat2_ref, w11_ref, w12_ref, w21_ref, w22_ref,
               f1_ref, f2_ref):
    f1_ref[...] = _dotT(_dotT(feat1_ref[...], w11_ref[...]), w12_ref[...])
    f2_ref[...] = _dotT(_dotT(feat2_ref[...], w21_ref[...]), w22_ref[...])


EBLK = 2000


def _edge_features(feature1, feature2, f1_W1, f1_W2, f2_W1, f2_W2):
    grid = (E // EBLK,)
    return pl.pallas_call(
        _edge_body,
        grid=grid,
        in_specs=[
            pl.BlockSpec((EBLK, F1), lambda i: (i, 0)),
            pl.BlockSpec((EBLK, F2), lambda i: (i, 0)),
            pl.BlockSpec((MID, F1), lambda i: (0, 0)),
            pl.BlockSpec((H, MID), lambda i: (0, 0)),
            pl.BlockSpec((MID, F2), lambda i: (0, 0)),
            pl.BlockSpec((H, MID), lambda i: (0, 0)),
        ],
        out_specs=[
            pl.BlockSpec((EBLK, H), lambda i: (i, 0)),
            pl.BlockSpec((EBLK, H), lambda i: (i, 0)),
        ],
        out_shape=[
            jax.ShapeDtypeStruct((E, H), jnp.float32),
            jax.ShapeDtypeStruct((E, H), jnp.float32),
        ],
    )(feature1, feature2, f1_W1, f1_W2, f2_W1, f2_W2)


def _sc_body(x1_hbm, f1_hbm, f2_hbm, src_hbm, dst_hbm,
             out1_hbm, out2_hbm,
             agg_sp,
             srcv, dstv, fv, xg, mv, zbuf,
             gsem):
    c = lax.axis_index("c")
    s = lax.axis_index("s")
    wid = s * NUM_CORES + c
    row0 = s * ROWS

    def _zb(i, _):
        r = i // (H // 16)
        l = (i % (H // 16)) * 16
        zbuf[r, pl.ds(l, 16)] = jnp.zeros((16,), jnp.float32)
        return 0
    lax.fori_loop(0, ZROWS * (H // 16), _zb, 0)

    for f_hbm, out_hbm in ((f1_hbm, out1_hbm), (f2_hbm, out2_hbm)):
        for j in range(ROWS // ZROWS):
            pltpu.sync_copy(zbuf, agg_sp.at[pl.ds(row0 + j * ZROWS, ZROWS)])
        plsc.subcore_barrier()

        def _chunk(k, _):
            e0 = wid * EW + k * CHUNK
            pltpu.sync_copy(src_hbm.at[pl.ds(e0, CHUNK)], srcv)
            pltpu.sync_copy(dst_hbm.at[pl.ds(e0, CHUNK)], dstv)
            pltpu.sync_copy(f_hbm.at[pl.ds(e0, CHUNK)], fv)
            pltpu.async_copy(x1_hbm.at[srcv], xg, gsem).wait()

            def _mul(i, _):
                r = i // (H // 16)
                l = (i % (H // 16)) * 16
                mv[r, pl.ds(l, 16)] = xg[r, pl.ds(l, 16)] * fv[r, pl.ds(l, 16)]
                return 0
            lax.fori_loop(0, CHUNK * (H // 16), _mul, 0)

            pltpu.sync_copy(mv, agg_sp.at[dstv], add=True)
            return 0
        lax.fori_loop(0, NCHUNK, _chunk, 0)
        plsc.subcore_barrier()

        pltpu.sync_copy(agg_sp.at[pl.ds(row0, ROWS)],
                        out_hbm.at[c, pl.ds(row0, ROWS)])


def _sc_aggregate(x1, f1, f2, src, dst):
    mesh = plsc.VectorSubcoreMesh(core_axis_name="c", subcore_axis_name="s")
    fn = pl.kernel(
        _sc_body,
        out_type=[
            jax.ShapeDtypeStruct((NUM_CORES, NP, H), jnp.float32),
            jax.ShapeDtypeStruct((NUM_CORES, NP, H), jnp.float32),
        ],
        mesh=mesh,
        scratch_types=[
            pltpu.VMEM_SHARED((NP, H), jnp.float32),
            pltpu.VMEM((CHUNK,), jnp.int32),
            pltpu.VMEM((CHUNK,), jnp.int32),
            pltpu.VMEM((CHUNK, H), jnp.float32),
            pltpu.VMEM((CHUNK, H), jnp.float32),
            pltpu.VMEM((CHUNK, H), jnp.float32),
            pltpu.VMEM((ZROWS, H), jnp.float32),
            pltpu.SemaphoreType.DMA,
        ],
    )
    return fn(x1, f1, f2, src, dst)


NBLK = 1024


def _node1_body(parts1_ref, parts2_ref, x1_ref,
                c1_Wrel_ref, c1_brel_ref, c1_Wroot_ref,
                c2_Wrel_ref, c2_brel_ref, c2_Wroot_ref,
                lin1_W_ref, lin1_b_ref, lin2_W_ref, lin2_b_ref,
                cat_W_ref, cat_b_ref, lins_W_ref, lins_b_ref,
                h_ref):
    x1 = x1_ref[...]
    parts1 = parts1_ref[...]
    parts2 = parts2_ref[...]
    agg1 = parts1[0] + parts1[1]
    agg2 = parts2[0] + parts2[1]

    conv1 = _dotT(agg1, c1_Wrel_ref[...]) + c1_brel_ref[...] \
        + _dotT(x1, c1_Wroot_ref[...])
    conv2 = _dotT(agg2, c2_Wrel_ref[...]) + c2_brel_ref[...] \
        + _dotT(x1, c2_Wroot_ref[...])
    h1 = _swish(_dotT(conv1, lin1_W_ref[...]) + lin1_b_ref[...])
    h2 = _swish(_dotT(conv2, lin2_W_ref[...]) + lin2_b_ref[...])

    cat_W = cat_W_ref[...]
    h = _dotT(h1, cat_W[:, :H]) + _dotT(h2, cat_W[:, H:]) + cat_b_ref[...] + x1

    lins_W = lins_W_ref[...]
    lins_b = lins_b_ref[...]
    for i in range(NL):
        h = _swish(_dotT(h, lins_W[i]) + lins_b[i][None, :]) + h
    h_ref[...] = h


def _node2_body(h_ref, batch_ref, gn_w_ref, gn_b_ref, gn_ms_ref,
                final_W_ref, final_b_ref, out_ref):
    h = h_ref[...][:N]
    gids = jax.lax.broadcasted_iota(jnp.int32, (N, NG), 1)
    oh = (batch_ref[...] == gids).astype(jnp.float32)
    cnt = jnp.maximum(jnp.sum(oh, axis=0), 1.0)
    sums = lax.dot_general(oh, h, (((0,), (0,)), ((), ())),
                           preferred_element_type=jnp.float32)
    mean = sums / cnt[:, None]
    mean_n = lax.dot_general(oh, mean, (((1,), (0,)), ((), ())),
                             preferred_element_type=jnp.float32)
    outh = h - mean_n * gn_ms_ref[...]
    var = lax.dot_general(oh, outh * outh, (((0,), (0,)), ((), ())),
                          preferred_element_type=jnp.float32) / cnt[:, None]
    std = jnp.sqrt(var + 1e-5)
    std_n = lax.dot_general(oh, std, (((1,), (0,)), ((), ())),
                            preferred_element_type=jnp.float32)
    hn = gn_w_ref[...] * outh / std_n + gn_b_ref[...]
    out_ref[...] = _dotT(hn, final_W_ref[...]) + final_b_ref[...]


def _node_pipeline(parts1, parts2, x1, batch2d, w1, w2):
    h = pl.pallas_call(
        _node1_body,
        grid=(NP // NBLK,),
        in_specs=[
            pl.BlockSpec((NUM_CORES, NBLK, H), lambda i: (0, i, 0)),
            pl.BlockSpec((NUM_CORES, NBLK, H), lambda i: (0, i, 0)),
            pl.BlockSpec((NBLK, H), lambda i: (i, 0)),
        ] + [pl.BlockSpec(w.shape, lambda i, n=len(w.shape): (0,) * n)
             for w in w1],
        out_specs=pl.BlockSpec((NBLK, H), lambda i: (i, 0)),
        out_shape=jax.ShapeDtypeStruct((NP, H), jnp.float32),
    )(parts1, parts2, x1, *w1)
    return pl.pallas_call(
        _node2_body,
        out_shape=jax.ShapeDtypeStruct((N, H), jnp.float32),
    )(h, batch2d, *w2)


def kernel(x, feature1, feature2, edge_index, batch, lin_W, lin_b, f1_W1,
           f1_W2, f2_W1, f2_W2, c1_Wrel, c1_brel, c1_Wroot, c2_Wrel, c2_brel,
           c2_Wroot, lin1_W, lin1_b, lin2_W, lin2_b, cat_W, cat_b, lins_W,
           lins_b, gn_w, gn_b, gn_ms, final_W, final_b):
    src = edge_index[0]
    dst = edge_index[1]

    x_pad = jnp.pad(x, ((0, NP - N), (0, 0)))
    x1 = pl.pallas_call(
        _x1_body,
        out_shape=jax.ShapeDtypeStruct((NP, H), jnp.float32),
    )(x_pad, lin_W, lin_b.reshape(1, H))

    f1, f2 = _edge_features(feature1, feature2, f1_W1, f1_W2, f2_W1, f2_W2)

    parts1, parts2 = _sc_aggregate(x1, f1, f2, src, dst)

    w1 = (
        c1_Wrel, c1_brel.reshape(1, H), c1_Wroot,
        c2_Wrel, c2_brel.reshape(1, H), c2_Wroot,
        lin1_W, lin1_b.reshape(1, H), lin2_W, lin2_b.reshape(1, H),
        cat_W, cat_b.reshape(1, H), lins_W, lins_b,
    )
    w2 = (
        gn_w.reshape(1, H), gn_b.reshape(1, H), gn_ms.reshape(1, H),
        final_W, final_b.reshape(1, H),
    )
    return _node_pipeline(parts1, parts2, x1, batch.reshape(N, 1), w1, w2)


# speedup vs baseline: 2.0162x; 1.5403x over previous
import jax
import jax.numpy as jnp
from jax import lax
from jax.experimental import pallas as pl
from jax.experimental.pallas import tpu as pltpu
from jax.experimental.pallas import tpu_sc as plsc

N = 10000
NP = 10240
E = 320000
H = 128
MID = 64
F1 = 147
F2 = 21
NL = 4
NG = 64

NUM_CORES = 2
NUM_SUBCORES = 16
NW = NUM_CORES * NUM_SUBCORES
EW = E // NW
CHUNK = 40
NCHUNK = EW // CHUNK
ROWS = NP // NUM_SUBCORES
ZROWS = 128


def _dotT(a, w):
    return lax.dot_general(a, w, (((1,), (1,)), ((), ())),
                           preferred_element_type=jnp.float32)


def _swish(v):
    return v * jax.nn.sigmoid(v)


def _x1_body(x_ref, w_ref, b_ref, o_ref):
    o_ref[...] = _swish(_dotT(x_ref[...], w_ref[...]) + b_ref[...])


def _edge_body(feat1_ref, feat2_ref, w11_ref, w12_ref, w21_ref, w22_ref,
               f1_ref, f2_ref):
    f1_ref[...] = _dotT(_dotT(feat1_ref[...], w11_ref[...]), w12_ref[...])
    f2_ref[...] = _dotT(_dotT(feat2_ref[...], w21_ref[...]), w22_ref[...])


EBLK = 2000


def _edge_features(feature1, feature2, f1_W1, f1_W2, f2_W1, f2_W2):
    grid = (E // EBLK,)
    return pl.pallas_call(
        _edge_body,
        grid=grid,
        in_specs=[
            pl.BlockSpec((EBLK, F1), lambda i: (i, 0)),
            pl.BlockSpec((EBLK, F2), lambda i: (i, 0)),
            pl.BlockSpec((MID, F1), lambda i: (0, 0)),
            pl.BlockSpec((H, MID), lambda i: (0, 0)),
            pl.BlockSpec((MID, F2), lambda i: (0, 0)),
            pl.BlockSpec((H, MID), lambda i: (0, 0)),
        ],
        out_specs=[
            pl.BlockSpec((EBLK, H), lambda i: (i, 0)),
            pl.BlockSpec((EBLK, H), lambda i: (i, 0)),
        ],
        out_shape=[
            jax.ShapeDtypeStruct((E, H), jnp.float32),
            jax.ShapeDtypeStruct((E, H), jnp.float32),
        ],
    )(feature1, feature2, f1_W1, f1_W2, f2_W1, f2_W2)


def _sc_body(x1_hbm, f1_hbm, f2_hbm, src_hbm, dst_hbm,
             out1_hbm, out2_hbm,
             agg_sp,
             srcv0, srcv1, dstv0, dstv1, fv0, fv1,
             xg0, xg1, mv0, mv1, zbuf,
             sl0, sl1, sg0, sg1):
    c = lax.axis_index("c")
    s = lax.axis_index("s")
    wid = s * NUM_CORES + c
    row0 = s * ROWS

    srcv = (srcv0, srcv1)
    dstv = (dstv0, dstv1)
    fv = (fv0, fv1)
    xg = (xg0, xg1)
    mv = (mv0, mv1)
    sl = (sl0, sl1)
    sg = (sg0, sg1)

    def _zb(i, _):
        r = i // (H // 16)
        l = (i % (H // 16)) * 16
        zbuf[r, pl.ds(l, 16)] = jnp.zeros((16,), jnp.float32)
        return 0
    lax.fori_loop(0, ZROWS * (H // 16), _zb, 0)

    for f_hbm, out_hbm in ((f1_hbm, out1_hbm), (f2_hbm, out2_hbm)):
        for j in range(ROWS // ZROWS):
            pltpu.sync_copy(zbuf, agg_sp.at[pl.ds(row0 + j * ZROWS, ZROWS)])
        plsc.subcore_barrier()

        def load_start(k, b):
            e0 = wid * EW + k * CHUNK
            pltpu.async_copy(src_hbm.at[pl.ds(e0, CHUNK)], srcv[b], sl[b])
            pltpu.async_copy(dst_hbm.at[pl.ds(e0, CHUNK)], dstv[b], sl[b])
            pltpu.async_copy(f_hbm.at[pl.ds(e0, CHUNK)], fv[b], sl[b])

        def load_wait(b):
            pltpu.make_async_copy(src_hbm.at[pl.ds(0, CHUNK)], srcv[b], sl[b]).wait()
            pltpu.make_async_copy(dst_hbm.at[pl.ds(0, CHUNK)], dstv[b], sl[b]).wait()
            pltpu.make_async_copy(f_hbm.at[pl.ds(0, CHUNK)], fv[b], sl[b]).wait()

        def gather_start(b):
            pltpu.async_copy(x1_hbm.at[srcv[b]], xg[b], sg[b])

        def gather_wait(b):
            pltpu.make_async_copy(x1_hbm.at[pl.ds(0, CHUNK)], xg[b], sg[b]).wait()

        load_start(0, 0)
        load_wait(0)
        gather_start(0)
        load_start(1, 1)

        def _pair(kk, _):
            for b in (0, 1):
                k = 2 * kk + b
                gather_wait(b)

                @pl.when(k + 1 < NCHUNK)
                def _():
                    load_wait(1 - b)
                    gather_start(1 - b)

                def _mul(r, _):
                    for l in range(H // 16):
                        mv[b][r, pl.ds(l * 16, 16)] = (
                            xg[b][r, pl.ds(l * 16, 16)]
                            * fv[b][r, pl.ds(l * 16, 16)])
                    return 0
                lax.fori_loop(0, CHUNK, _mul, 0)

                pltpu.sync_copy(mv[b], agg_sp.at[dstv[b]], add=True)

                @pl.when(k + 2 < NCHUNK)
                def _():
                    load_start(k + 2, b)
            return 0

        lax.fori_loop(0, NCHUNK // 2, _pair, 0)
        plsc.subcore_barrier()

        pltpu.sync_copy(agg_sp.at[pl.ds(row0, ROWS)],
                        out_hbm.at[c, pl.ds(row0, ROWS)])


def _sc_aggregate(x1, f1, f2, src, dst):
    mesh = plsc.VectorSubcoreMesh(core_axis_name="c", subcore_axis_name="s")
    fn = pl.kernel(
        _sc_body,
        out_type=[
            jax.ShapeDtypeStruct((NUM_CORES, NP, H), jnp.float32),
            jax.ShapeDtypeStruct((NUM_CORES, NP, H), jnp.float32),
        ],
        mesh=mesh,
        scratch_types=[
            pltpu.VMEM_SHARED((NP, H), jnp.float32),
            pltpu.VMEM((CHUNK,), jnp.int32),
            pltpu.VMEM((CHUNK,), jnp.int32),
            pltpu.VMEM((CHUNK,), jnp.int32),
            pltpu.VMEM((CHUNK,), jnp.int32),
            pltpu.VMEM((CHUNK, H), jnp.float32),
            pltpu.VMEM((CHUNK, H), jnp.float32),
            pltpu.VMEM((CHUNK, H), jnp.float32),
            pltpu.VMEM((CHUNK, H), jnp.float32),
            pltpu.VMEM((CHUNK, H), jnp.float32),
            pltpu.VMEM((CHUNK, H), jnp.float32),
            pltpu.VMEM((ZROWS, H), jnp.float32),
            pltpu.SemaphoreType.DMA,
            pltpu.SemaphoreType.DMA,
            pltpu.SemaphoreType.DMA,
            pltpu.SemaphoreType.DMA,
        ],
    )
    return fn(x1, f1, f2, src, dst)


NBLK = 1024


def _node1_body(parts1_ref, parts2_ref, x1_ref,
                c1_Wrel_ref, c1_brel_ref, c1_Wroot_ref,
                c2_Wrel_ref, c2_brel_ref, c2_Wroot_ref,
                lin1_W_ref, lin1_b_ref, lin2_W_ref, lin2_b_ref,
                cat_W_ref, cat_b_ref, lins_W_ref, lins_b_ref,
                h_ref):
    x1 = x1_ref[...]
    parts1 = parts1_ref[...]
    parts2 = parts2_ref[...]
    agg1 = parts1[0] + parts1[1]
    agg2 = parts2[0] + parts2[1]

    conv1 = _dotT(agg1, c1_Wrel_ref[...]) + c1_brel_ref[...] \
        + _dotT(x1, c1_Wroot_ref[...])
    conv2 = _dotT(agg2, c2_Wrel_ref[...]) + c2_brel_ref[...] \
        + _dotT(x1, c2_Wroot_ref[...])
    h1 = _swish(_dotT(conv1, lin1_W_ref[...]) + lin1_b_ref[...])
    h2 = _swish(_dotT(conv2, lin2_W_ref[...]) + lin2_b_ref[...])

    cat_W = cat_W_ref[...]
    h = _dotT(h1, cat_W[:, :H]) + _dotT(h2, cat_W[:, H:]) + cat_b_ref[...] + x1

    lins_W = lins_W_ref[...]
    lins_b = lins_b_ref[...]
    for i in range(NL):
        h = _swish(_dotT(h, lins_W[i]) + lins_b[i][None, :]) + h
    h_ref[...] = h


def _node2_body(h_ref, batch_ref, gn_w_ref, gn_b_ref, gn_ms_ref,
                final_W_ref, final_b_ref, out_ref):
    h = h_ref[...][:N]
    gids = jax.lax.broadcasted_iota(jnp.int32, (N, NG), 1)
    oh = (batch_ref[...] == gids).astype(jnp.float32)
    cnt = jnp.maximum(jnp.sum(oh, axis=0), 1.0)
    sums = lax.dot_general(oh, h, (((0,), (0,)), ((), ())),
                           preferred_element_type=jnp.float32)
    mean = sums / cnt[:, None]
    mean_n = lax.dot_general(oh, mean, (((1,), (0,)), ((), ())),
                             preferred_element_type=jnp.float32)
    outh = h - mean_n * gn_ms_ref[...]
    var = lax.dot_general(oh, outh * outh, (((0,), (0,)), ((), ())),
                          preferred_element_type=jnp.float32) / cnt[:, None]
    std = jnp.sqrt(var + 1e-5)
    std_n = lax.dot_general(oh, std, (((1,), (0,)), ((), ())),
                            preferred_element_type=jnp.float32)
    hn = gn_w_ref[...] * outh / std_n + gn_b_ref[...]
    out_ref[...] = _dotT(hn, final_W_ref[...]) + final_b_ref[...]


def _node_pipeline(parts1, parts2, x1, batch2d, w1, w2):
    h = pl.pallas_call(
        _node1_body,
        grid=(NP // NBLK,),
        in_specs=[
            pl.BlockSpec((NUM_CORES, NBLK, H), lambda i: (0, i, 0)),
            pl.BlockSpec((NUM_CORES, NBLK, H), lambda i: (0, i, 0)),
            pl.BlockSpec((NBLK, H), lambda i: (i, 0)),
        ] + [pl.BlockSpec(w.shape, lambda i, n=len(w.shape): (0,) * n)
             for w in w1],
        out_specs=pl.BlockSpec((NBLK, H), lambda i: (i, 0)),
        out_shape=jax.ShapeDtypeStruct((NP, H), jnp.float32),
    )(parts1, parts2, x1, *w1)
    return pl.pallas_call(
        _node2_body,
        out_shape=jax.ShapeDtypeStruct((N, H), jnp.float32),
    )(h, batch2d, *w2)


def kernel(x, feature1, feature2, edge_index, batch, lin_W, lin_b, f1_W1,
           f1_W2, f2_W1, f2_W2, c1_Wrel, c1_brel, c1_Wroot, c2_Wrel, c2_brel,
           c2_Wroot, lin1_W, lin1_b, lin2_W, lin2_b, cat_W, cat_b, lins_W,
           lins_b, gn_w, gn_b, gn_ms, final_W, final_b):
    src = edge_index[0]
    dst = edge_index[1]

    x_pad = jnp.pad(x, ((0, NP - N), (0, 0)))
    x1 = pl.pallas_call(
        _x1_body,
        out_shape=jax.ShapeDtypeStruct((NP, H), jnp.float32),
    )(x_pad, lin_W, lin_b.reshape(1, H))

    f1, f2 = _edge_features(feature1, feature2, f1_W1, f1_W2, f2_W1, f2_W2)

    parts1, parts2 = _sc_aggregate(x1, f1, f2, src, dst)

    w1 = (
        c1_Wrel, c1_brel.reshape(1, H), c1_Wroot,
        c2_Wrel, c2_brel.reshape(1, H), c2_Wroot,
        lin1_W, lin1_b.reshape(1, H), lin2_W, lin2_b.reshape(1, H),
        cat_W, cat_b.reshape(1, H), lins_W, lins_b,
    )
    w2 = (
        gn_w.reshape(1, H), gn_b.reshape(1, H), gn_ms.reshape(1, H),
        final_W, final_b.reshape(1, H),
    )
    return _node_pipeline(parts1, parts2, x1, batch.reshape(N, 1), w1, w2)


# speedup vs baseline: 2.0443x; 1.0140x over previous
import jax
import jax.numpy as jnp
from jax import lax
from jax.experimental import pallas as pl
from jax.experimental.pallas import tpu as pltpu
from jax.experimental.pallas import tpu_sc as plsc

N = 10000
NP = 10240
E = 320000
H = 128
MID = 64
F1 = 147
F2 = 21
NL = 4
NG = 64

NUM_CORES = 2
NUM_SUBCORES = 16
NW = NUM_CORES * NUM_SUBCORES
EW = E // NW
CHUNK = 40
NCHUNK = EW // CHUNK
ROWS = NP // NUM_SUBCORES
ZROWS = 128


def _dotT(a, w):
    return lax.dot_general(a, w, (((1,), (1,)), ((), ())),
                           preferred_element_type=jnp.float32)


def _swish(v):
    return v * jax.nn.sigmoid(v)


def _x1_body(x_ref, w_ref, b_ref, o_ref):
    o_ref[...] = _swish(_dotT(x_ref[...], w_ref[...]) + b_ref[...])


def _pack_bf16(y):
    u = lax.bitcast_convert_type(y.astype(jnp.bfloat16), jnp.uint16)
    lo = jnp.concatenate([u[:, 32 * g:32 * g + 16] for g in range(4)], axis=1)
    hi = jnp.concatenate([u[:, 32 * g + 16:32 * g + 32] for g in range(4)],
                         axis=1)
    w = lo.astype(jnp.uint32) | (hi.astype(jnp.uint32) << 16)
    return lax.bitcast_convert_type(w, jnp.int32)


def _edge_body(feat1_ref, feat2_ref, w11_ref, w12_ref, w21_ref, w22_ref,
               f1_ref, f2_ref):
    f1_ref[...] = _pack_bf16(_dotT(_dotT(feat1_ref[...], w11_ref[...]),
                                   w12_ref[...]))
    f2_ref[...] = _pack_bf16(_dotT(_dotT(feat2_ref[...], w21_ref[...]),
                                   w22_ref[...]))


EBLK = 2000


def _edge_features(feature1, feature2, f1_W1, f1_W2, f2_W1, f2_W2):
    grid = (E // EBLK,)
    return pl.pallas_call(
        _edge_body,
        grid=grid,
        in_specs=[
            pl.BlockSpec((EBLK, F1), lambda i: (i, 0)),
            pl.BlockSpec((EBLK, F2), lambda i: (i, 0)),
            pl.BlockSpec((MID, F1), lambda i: (0, 0)),
            pl.BlockSpec((H, MID), lambda i: (0, 0)),
            pl.BlockSpec((MID, F2), lambda i: (0, 0)),
            pl.BlockSpec((H, MID), lambda i: (0, 0)),
        ],
        out_specs=[
            pl.BlockSpec((EBLK, H // 2), lambda i: (i, 0)),
            pl.BlockSpec((EBLK, H // 2), lambda i: (i, 0)),
        ],
        out_shape=[
            jax.ShapeDtypeStruct((E, H // 2), jnp.int32),
            jax.ShapeDtypeStruct((E, H // 2), jnp.int32),
        ],
    )(feature1, feature2, f1_W1, f1_W2, f2_W1, f2_W2)


def _sc_body(x1_hbm, f1_hbm, f2_hbm, src_hbm, dst_hbm,
             out1_hbm, out2_hbm,
             agg_sp,
             srcv0, srcv1, dstv0, dstv1, fv0, fv1,
             xg0, xg1, mv0, mv1, zbuf,
             sl0, sl1, sg0, sg1):
    c = lax.axis_index("c")
    s = lax.axis_index("s")
    wid = s * NUM_CORES + c
    row0 = s * ROWS

    srcv = (srcv0, srcv1)
    dstv = (dstv0, dstv1)
    fv = (fv0, fv1)
    xg = (xg0, xg1)
    mv = (mv0, mv1)
    sl = (sl0, sl1)
    sg = (sg0, sg1)

    def _zb(i, _):
        r = i // (H // 16)
        l = (i % (H // 16)) * 16
        zbuf[r, pl.ds(l, 16)] = jnp.zeros((16,), jnp.float32)
        return 0
    lax.fori_loop(0, ZROWS * (H // 16), _zb, 0)

    for f_hbm, out_hbm in ((f1_hbm, out1_hbm), (f2_hbm, out2_hbm)):
        for j in range(ROWS // ZROWS):
            pltpu.sync_copy(zbuf, agg_sp.at[pl.ds(row0 + j * ZROWS, ZROWS)])
        plsc.subcore_barrier()

        def load_start(k, b):
            e0 = wid * EW + k * CHUNK
            pltpu.async_copy(src_hbm.at[pl.ds(e0, CHUNK)], srcv[b], sl[b])
            pltpu.async_copy(dst_hbm.at[pl.ds(e0, CHUNK)], dstv[b], sl[b])
            pltpu.async_copy(f_hbm.at[pl.ds(e0, CHUNK)], fv[b], sl[b])

        def load_wait(b):
            pltpu.make_async_copy(src_hbm.at[pl.ds(0, CHUNK)], srcv[b], sl[b]).wait()
            pltpu.make_async_copy(dst_hbm.at[pl.ds(0, CHUNK)], dstv[b], sl[b]).wait()
            pltpu.make_async_copy(f_hbm.at[pl.ds(0, CHUNK)], fv[b], sl[b]).wait()


        def gather_start(b):
            pltpu.async_copy(x1_hbm.at[srcv[b]], xg[b], sg[b])

        def gather_wait(b):
            pltpu.make_async_copy(x1_hbm.at[pl.ds(0, CHUNK)], xg[b], sg[b]).wait()

        load_start(0, 0)
        load_wait(0)
        gather_start(0)
        load_start(1, 1)

        def _pair(kk, _):
            for b in (0, 1):
                k = 2 * kk + b
                gather_wait(b)

                @pl.when(k + 1 < NCHUNK)
                def _():
                    load_wait(1 - b)
                    gather_start(1 - b)

                def _mul(r, _):
                    for g in range(4):
                        w = fv[b][r, pl.ds(g * 16, 16)]
                        lo = lax.bitcast_convert_type(w << 16, jnp.float32)
                        hi = lax.bitcast_convert_type(
                            w & jnp.int32(-65536), jnp.float32)
                        mv[b][r, pl.ds(g * 32, 16)] = (
                            xg[b][r, pl.ds(g * 32, 16)] * lo)
                        mv[b][r, pl.ds(g * 32 + 16, 16)] = (
                            xg[b][r, pl.ds(g * 32 + 16, 16)] * hi)
                    return 0
                lax.fori_loop(0, CHUNK, _mul, 0)

                pltpu.sync_copy(mv[b], agg_sp.at[dstv[b]], add=True)

                @pl.when(k + 2 < NCHUNK)
                def _():
                    load_start(k + 2, b)
            return 0

        lax.fori_loop(0, NCHUNK // 2, _pair, 0)
        plsc.subcore_barrier()

        pltpu.sync_copy(agg_sp.at[pl.ds(row0, ROWS)],
                        out_hbm.at[c, pl.ds(row0, ROWS)])


def _sc_aggregate(x1, f1, f2, src, dst):
    mesh = plsc.VectorSubcoreMesh(core_axis_name="c", subcore_axis_name="s")
    fn = pl.kernel(
        _sc_body,
        out_type=[
            jax.ShapeDtypeStruct((NUM_CORES, NP, H), jnp.float32),
            jax.ShapeDtypeStruct((NUM_CORES, NP, H), jnp.float32),
        ],
        mesh=mesh,
        scratch_types=[
            pltpu.VMEM_SHARED((NP, H), jnp.float32),
            pltpu.VMEM((CHUNK,), jnp.int32),
            pltpu.VMEM((CHUNK,), jnp.int32),
            pltpu.VMEM((CHUNK,), jnp.int32),
            pltpu.VMEM((CHUNK,), jnp.int32),
            pltpu.VMEM((CHUNK, H // 2), jnp.int32),
            pltpu.VMEM((CHUNK, H // 2), jnp.int32),
            pltpu.VMEM((CHUNK, H), jnp.float32),
            pltpu.VMEM((CHUNK, H), jnp.float32),
            pltpu.VMEM((CHUNK, H), jnp.float32),
            pltpu.VMEM((CHUNK, H), jnp.float32),
            pltpu.VMEM((ZROWS, H), jnp.float32),
            pltpu.SemaphoreType.DMA,
            pltpu.SemaphoreType.DMA,
            pltpu.SemaphoreType.DMA,
            pltpu.SemaphoreType.DMA,
        ],
    )
    return fn(x1, f1, f2, src, dst)


NBLK = 1024


def _node1_body(parts1_ref, parts2_ref, x1_ref,
                c1_Wrel_ref, c1_brel_ref, c1_Wroot_ref,
                c2_Wrel_ref, c2_brel_ref, c2_Wroot_ref,
                lin1_W_ref, lin1_b_ref, lin2_W_ref, lin2_b_ref,
                cat_W_ref, cat_b_ref, lins_W_ref, lins_b_ref,
                h_ref):
    x1 = x1_ref[...]
    parts1 = parts1_ref[...]
    parts2 = parts2_ref[...]
    agg1 = parts1[0] + parts1[1]
    agg2 = parts2[0] + parts2[1]

    conv1 = _dotT(agg1, c1_Wrel_ref[...]) + c1_brel_ref[...] \
        + _dotT(x1, c1_Wroot_ref[...])
    conv2 = _dotT(agg2, c2_Wrel_ref[...]) + c2_brel_ref[...] \
        + _dotT(x1, c2_Wroot_ref[...])
    h1 = _swish(_dotT(conv1, lin1_W_ref[...]) + lin1_b_ref[...])
    h2 = _swish(_dotT(conv2, lin2_W_ref[...]) + lin2_b_ref[...])

    cat_W = cat_W_ref[...]
    h = _dotT(h1, cat_W[:, :H]) + _dotT(h2, cat_W[:, H:]) + cat_b_ref[...] + x1

    lins_W = lins_W_ref[...]
    lins_b = lins_b_ref[...]
    for i in range(NL):
        h = _swish(_dotT(h, lins_W[i]) + lins_b[i][None, :]) + h
    h_ref[...] = h


def _node2_body(h_ref, batch_ref, gn_w_ref, gn_b_ref, gn_ms_ref,
                final_W_ref, final_b_ref, out_ref):
    h = h_ref[...][:N]
    gids = jax.lax.broadcasted_iota(jnp.int32, (N, NG), 1)
    oh = (batch_ref[...] == gids).astype(jnp.float32)
    cnt = jnp.maximum(jnp.sum(oh, axis=0), 1.0)
    sums = lax.dot_general(oh, h, (((0,), (0,)), ((), ())),
                           preferred_element_type=jnp.float32)
    mean = sums / cnt[:, None]
    mean_n = lax.dot_general(oh, mean, (((1,), (0,)), ((), ())),
                             preferred_element_type=jnp.float32)
    outh = h - mean_n * gn_ms_ref[...]
    var = lax.dot_general(oh, outh * outh, (((0,), (0,)), ((), ())),
                          preferred_element_type=jnp.float32) / cnt[:, None]
    std = jnp.sqrt(var + 1e-5)
    std_n = lax.dot_general(oh, std, (((1,), (0,)), ((), ())),
                            preferred_element_type=jnp.float32)
    hn = gn_w_ref[...] * outh / std_n + gn_b_ref[...]
    out_ref[...] = _dotT(hn, final_W_ref[...]) + final_b_ref[...]


def _node_pipeline(parts1, parts2, x1, batch2d, w1, w2):
    h = pl.pallas_call(
        _node1_body,
        grid=(NP // NBLK,),
        in_specs=[
            pl.BlockSpec((NUM_CORES, NBLK, H), lambda i: (0, i, 0)),
            pl.BlockSpec((NUM_CORES, NBLK, H), lambda i: (0, i, 0)),
            pl.BlockSpec((NBLK, H), lambda i: (i, 0)),
        ] + [pl.BlockSpec(w.shape, lambda i, n=len(w.shape): (0,) * n)
             for w in w1],
        out_specs=pl.BlockSpec((NBLK, H), lambda i: (i, 0)),
        out_shape=jax.ShapeDtypeStruct((NP, H), jnp.float32),
    )(parts1, parts2, x1, *w1)
    return pl.pallas_call(
        _node2_body,
        out_shape=jax.ShapeDtypeStruct((N, H), jnp.float32),
    )(h, batch2d, *w2)


def kernel(x, feature1, feature2, edge_index, batch, lin_W, lin_b, f1_W1,
           f1_W2, f2_W1, f2_W2, c1_Wrel, c1_brel, c1_Wroot, c2_Wrel, c2_brel,
           c2_Wroot, lin1_W, lin1_b, lin2_W, lin2_b, cat_W, cat_b, lins_W,
           lins_b, gn_w, gn_b, gn_ms, final_W, final_b):
    src = edge_index[0]
    dst = edge_index[1]

    x_pad = jnp.pad(x, ((0, NP - N), (0, 0)))
    x1 = pl.pallas_call(
        _x1_body,
        out_shape=jax.ShapeDtypeStruct((NP, H), jnp.float32),
    )(x_pad, lin_W, lin_b.reshape(1, H))

    f1, f2 = _edge_features(feature1, feature2, f1_W1, f1_W2, f2_W1, f2_W2)

    parts1, parts2 = _sc_aggregate(x1, f1, f2, src, dst)

    w1 = (
        c1_Wrel, c1_brel.reshape(1, H), c1_Wroot,
        c2_Wrel, c2_brel.reshape(1, H), c2_Wroot,
        lin1_W, lin1_b.reshape(1, H), lin2_W, lin2_b.reshape(1, H),
        cat_W, cat_b.reshape(1, H), lins_W, lins_b,
    )
    w2 = (
        gn_w.reshape(1, H), gn_b.reshape(1, H), gn_ms.reshape(1, H),
        final_W, final_b.reshape(1, H),
    )
    return _node_pipeline(parts1, parts2, x1, batch.reshape(N, 1), w1, w2)


# speedup vs baseline: 2.6098x; 1.2766x over previous
import jax
import jax.numpy as jnp
from jax import lax
from jax.experimental import pallas as pl
from jax.experimental.pallas import tpu as pltpu
from jax.experimental.pallas import tpu_sc as plsc

N = 10000
NP = 10240
E = 320000
H = 128
MID = 64
F1 = 147
F2 = 21
NL = 4
NG = 64

NUM_CORES = 2
NUM_SUBCORES = 16
NW = NUM_CORES * NUM_SUBCORES
EW = E // NW
CHUNK = 40
NCHUNK = EW // CHUNK
NBUF = 4
NBG = 3
UNROLL = 12
ROWS = NP // NUM_SUBCORES
ZROWS = 40


def _dotT(a, w):
    return lax.dot_general(a, w, (((1,), (1,)), ((), ())),
                           preferred_element_type=jnp.float32)


def _swish(v):
    return v * jax.nn.sigmoid(v)


def _x1_body(x_ref, w_ref, b_ref, o_ref):
    o_ref[...] = _swish(_dotT(x_ref[...], w_ref[...]) + b_ref[...])


def _pack_bf16(y):
    u = lax.bitcast_convert_type(y.astype(jnp.bfloat16), jnp.uint16)
    lo = jnp.concatenate([u[:, 32 * g:32 * g + 16] for g in range(4)], axis=1)
    hi = jnp.concatenate([u[:, 32 * g + 16:32 * g + 32] for g in range(4)],
                         axis=1)
    w = lo.astype(jnp.uint32) | (hi.astype(jnp.uint32) << 16)
    return lax.bitcast_convert_type(w, jnp.int32)


def _dotTb(a, w):
    return lax.dot_general(a.astype(jnp.bfloat16), w.astype(jnp.bfloat16),
                           (((1,), (1,)), ((), ())),
                           preferred_element_type=jnp.float32)


def _edge_body(feat1_ref, feat2_ref, w11_ref, w12_ref, w21_ref, w22_ref,
               f1_ref, f2_ref):
    f1_ref[...] = _pack_bf16(_dotTb(_dotTb(feat1_ref[...], w11_ref[...]),
                                    w12_ref[...]))
    f2_ref[...] = _pack_bf16(_dotTb(_dotTb(feat2_ref[...], w21_ref[...]),
                                    w22_ref[...]))


EBLK = 2000


def _edge_features(feature1, feature2, f1_W1, f1_W2, f2_W1, f2_W2):
    grid = (E // EBLK,)
    return pl.pallas_call(
        _edge_body,
        grid=grid,
        in_specs=[
            pl.BlockSpec((EBLK, F1), lambda i: (i, 0)),
            pl.BlockSpec((EBLK, F2), lambda i: (i, 0)),
            pl.BlockSpec((MID, F1), lambda i: (0, 0)),
            pl.BlockSpec((H, MID), lambda i: (0, 0)),
            pl.BlockSpec((MID, F2), lambda i: (0, 0)),
            pl.BlockSpec((H, MID), lambda i: (0, 0)),
        ],
        out_specs=[
            pl.BlockSpec((EBLK, H // 2), lambda i: (i, 0)),
            pl.BlockSpec((EBLK, H // 2), lambda i: (i, 0)),
        ],
        out_shape=[
            jax.ShapeDtypeStruct((E, H // 2), jnp.int32),
            jax.ShapeDtypeStruct((E, H // 2), jnp.int32),
        ],
    )(feature1, feature2, f1_W1, f1_W2, f2_W1, f2_W2)


def _sc_body(x1_hbm, f1_hbm, f2_hbm, src_hbm, dst_hbm,
             out1_hbm, out2_hbm,
             agg_sp,
             srcv, dstv, fv, xg, mv, zbuf,
             sl, sg):
    c = lax.axis_index("c")
    s = lax.axis_index("s")
    wid = s * NUM_CORES + c
    row0 = s * ROWS

    def _zb(i, _):
        r = i // (H // 16)
        l = (i % (H // 16)) * 16
        zbuf[r, pl.ds(l, 16)] = jnp.zeros((16,), jnp.float32)
        return 0
    lax.fori_loop(0, ZROWS * (H // 16), _zb, 0)

    for f_hbm, out_hbm in ((f1_hbm, out1_hbm), (f2_hbm, out2_hbm)):
        for j in range(ROWS // ZROWS):
            pltpu.sync_copy(zbuf, agg_sp.at[pl.ds(row0 + j * ZROWS, ZROWS)])
        plsc.subcore_barrier()

        def load_start(k, b):
            e0 = wid * EW + k * CHUNK
            pltpu.async_copy(src_hbm.at[pl.ds(e0, CHUNK)], srcv[b], sl[b])
            pltpu.async_copy(dst_hbm.at[pl.ds(e0, CHUNK)], dstv[b], sl[b])
            pltpu.async_copy(f_hbm.at[pl.ds(e0, CHUNK)], fv[b], sl[b])

        def load_wait(b):
            pltpu.make_async_copy(src_hbm.at[pl.ds(0, CHUNK)], srcv[b], sl[b]).wait()
            pltpu.make_async_copy(dst_hbm.at[pl.ds(0, CHUNK)], dstv[b], sl[b]).wait()
            pltpu.make_async_copy(f_hbm.at[pl.ds(0, CHUNK)], fv[b], sl[b]).wait()

        def gather_start(lb, gb):
            pltpu.async_copy(x1_hbm.at[srcv[lb]], xg[gb], sg[gb])

        def gather_wait(gb):
            pltpu.make_async_copy(x1_hbm.at[pl.ds(0, CHUNK)], xg[gb], sg[gb]).wait()

        load_start(0, 0)
        load_start(1, 1)

        def _grp(jj, _):
            for p in range(UNROLL):
                j = UNROLL * jj + p
                lb = p % NBUF
                gb = p % NBG
                plb = (p + 2) % NBUF
                pgb = (p + 1) % NBG

                @pl.when(j < NCHUNK)
                def _():
                    load_wait(lb)
                    gather_start(lb, gb)

                @pl.when(j >= 2)
                def _():
                    gather_wait(pgb)

                    def _mul(r, _):
                        for g in range(4):
                            w = fv[plb][r, pl.ds(g * 16, 16)]
                            lo = lax.bitcast_convert_type(w << 16, jnp.float32)
                            hi = lax.bitcast_convert_type(
                                w & jnp.int32(-65536), jnp.float32)
                            mv[r, pl.ds(g * 32, 16)] = (
                                xg[pgb][r, pl.ds(g * 32, 16)] * lo)
                            mv[r, pl.ds(g * 32 + 16, 16)] = (
                                xg[pgb][r, pl.ds(g * 32 + 16, 16)] * hi)
                        return 0
                    lax.fori_loop(0, CHUNK, _mul, 0)

                    pltpu.sync_copy(mv, agg_sp.at[dstv[plb]], add=True)

                @pl.when(j + 2 < NCHUNK)
                def _():
                    load_start(j + 2, plb)
            return 0

        lax.fori_loop(0, (NCHUNK + 2) // UNROLL, _grp, 0)
        plsc.subcore_barrier()

        pltpu.sync_copy(agg_sp.at[pl.ds(row0, ROWS)],
                        out_hbm.at[c, pl.ds(row0, ROWS)])


def _sc_aggregate(x1, f1, f2, src, dst):
    mesh = plsc.VectorSubcoreMesh(core_axis_name="c", subcore_axis_name="s")
    fn = pl.kernel(
        _sc_body,
        out_type=[
            jax.ShapeDtypeStruct((NUM_CORES, NP, H), jnp.float32),
            jax.ShapeDtypeStruct((NUM_CORES, NP, H), jnp.float32),
        ],
        mesh=mesh,
        scratch_types=[
            pltpu.VMEM_SHARED((NP, H), jnp.float32),
            tuple(pltpu.VMEM((CHUNK,), jnp.int32) for _ in range(NBUF)),
            tuple(pltpu.VMEM((CHUNK,), jnp.int32) for _ in range(NBUF)),
            tuple(pltpu.VMEM((CHUNK, H // 2), jnp.int32) for _ in range(NBUF)),
            tuple(pltpu.VMEM((CHUNK, H), jnp.float32) for _ in range(NBG)),
            pltpu.VMEM((CHUNK, H), jnp.float32),
            pltpu.VMEM((ZROWS, H), jnp.float32),
            tuple(pltpu.SemaphoreType.DMA for _ in range(NBUF)),
            tuple(pltpu.SemaphoreType.DMA for _ in range(NBG)),
        ],
    )
    return fn(x1, f1, f2, src, dst)


NBLK = 1024


def _node1_body(parts1_ref, parts2_ref, x1_ref,
                c1_Wrel_ref, c1_brel_ref, c1_Wroot_ref,
                c2_Wrel_ref, c2_brel_ref, c2_Wroot_ref,
                lin1_W_ref, lin1_b_ref, lin2_W_ref, lin2_b_ref,
                cat_W_ref, cat_b_ref, lins_W_ref, lins_b_ref,
                h_ref):
    x1 = x1_ref[...]
    parts1 = parts1_ref[...]
    parts2 = parts2_ref[...]
    agg1 = parts1[0] + parts1[1]
    agg2 = parts2[0] + parts2[1]

    conv1 = _dotT(agg1, c1_Wrel_ref[...]) + c1_brel_ref[...] \
        + _dotT(x1, c1_Wroot_ref[...])
    conv2 = _dotT(agg2, c2_Wrel_ref[...]) + c2_brel_ref[...] \
        + _dotT(x1, c2_Wroot_ref[...])
    h1 = _swish(_dotT(conv1, lin1_W_ref[...]) + lin1_b_ref[...])
    h2 = _swish(_dotT(conv2, lin2_W_ref[...]) + lin2_b_ref[...])

    cat_W = cat_W_ref[...]
    h = _dotT(h1, cat_W[:, :H]) + _dotT(h2, cat_W[:, H:]) + cat_b_ref[...] + x1

    lins_W = lins_W_ref[...]
    lins_b = lins_b_ref[...]
    for i in range(NL):
        h = _swish(_dotT(h, lins_W[i]) + lins_b[i][None, :]) + h
    h_ref[...] = h


def _node2_body(h_ref, batch_ref, gn_w_ref, gn_b_ref, gn_ms_ref,
                final_W_ref, final_b_ref, out_ref):
    h = h_ref[...][:N]
    gids = jax.lax.broadcasted_iota(jnp.int32, (N, NG), 1)
    oh = (batch_ref[...] == gids).astype(jnp.float32)
    cnt = jnp.maximum(jnp.sum(oh, axis=0), 1.0)
    sums = lax.dot_general(oh, h, (((0,), (0,)), ((), ())),
                           preferred_element_type=jnp.float32)
    mean = sums / cnt[:, None]
    mean_n = lax.dot_general(oh, mean, (((1,), (0,)), ((), ())),
                             preferred_element_type=jnp.float32)
    outh = h - mean_n * gn_ms_ref[...]
    var = lax.dot_general(oh, outh * outh, (((0,), (0,)), ((), ())),
                          preferred_element_type=jnp.float32) / cnt[:, None]
    std = jnp.sqrt(var + 1e-5)
    std_n = lax.dot_general(oh, std, (((1,), (0,)), ((), ())),
                            preferred_element_type=jnp.float32)
    hn = gn_w_ref[...] * outh / std_n + gn_b_ref[...]
    out_ref[...] = _dotT(hn, final_W_ref[...]) + final_b_ref[...]


def _node_pipeline(parts1, parts2, x1, batch2d, w1, w2):
    h = pl.pallas_call(
        _node1_body,
        grid=(NP // NBLK,),
        in_specs=[
            pl.BlockSpec((NUM_CORES, NBLK, H), lambda i: (0, i, 0)),
            pl.BlockSpec((NUM_CORES, NBLK, H), lambda i: (0, i, 0)),
            pl.BlockSpec((NBLK, H), lambda i: (i, 0)),
        ] + [pl.BlockSpec(w.shape, lambda i, n=len(w.shape): (0,) * n)
             for w in w1],
        out_specs=pl.BlockSpec((NBLK, H), lambda i: (i, 0)),
        out_shape=jax.ShapeDtypeStruct((NP, H), jnp.float32),
    )(parts1, parts2, x1, *w1)
    return pl.pallas_call(
        _node2_body,
        out_shape=jax.ShapeDtypeStruct((N, H), jnp.float32),
    )(h, batch2d, *w2)


def kernel(x, feature1, feature2, edge_index, batch, lin_W, lin_b, f1_W1,
           f1_W2, f2_W1, f2_W2, c1_Wrel, c1_brel, c1_Wroot, c2_Wrel, c2_brel,
           c2_Wroot, lin1_W, lin1_b, lin2_W, lin2_b, cat_W, cat_b, lins_W,
           lins_b, gn_w, gn_b, gn_ms, final_W, final_b):
    src = edge_index[0]
    dst = edge_index[1]

    x_pad = jnp.pad(x, ((0, NP - N), (0, 0)))
    x1 = pl.pallas_call(
        _x1_body,
        out_shape=jax.ShapeDtypeStruct((NP, H), jnp.float32),
    )(x_pad, lin_W, lin_b.reshape(1, H))

    f1, f2 = _edge_features(feature1, feature2, f1_W1, f1_W2, f2_W1, f2_W2)

    parts1, parts2 = _sc_aggregate(x1, f1, f2, src, dst)

    w1 = (
        c1_Wrel, c1_brel.reshape(1, H), c1_Wroot,
        c2_Wrel, c2_brel.reshape(1, H), c2_Wroot,
        lin1_W, lin1_b.reshape(1, H), lin2_W, lin2_b.reshape(1, H),
        cat_W, cat_b.reshape(1, H), lins_W, lins_b,
    )
    w2 = (
        gn_w.reshape(1, H), gn_b.reshape(1, H), gn_ms.reshape(1, H),
        final_W, final_b.reshape(1, H),
    )
    return _node_pipeline(parts1, parts2, x1, batch.reshape(N, 1), w1, w2)


# speedup vs baseline: 2.7644x; 1.0592x over previous
import jax
import jax.numpy as jnp
from jax import lax
from jax.experimental import pallas as pl
from jax.experimental.pallas import tpu as pltpu
from jax.experimental.pallas import tpu_sc as plsc

N = 10000
NP = 10240
E = 320000
H = 128
MID = 64
F1 = 147
F2 = 21
NL = 4
NG = 64

NUM_CORES = 2
NUM_SUBCORES = 16
NW = NUM_CORES * NUM_SUBCORES
EW = E // NW
CHUNK = 40
NCHUNK = EW // CHUNK
NBUF = 4
NBG = 3
UNROLL = 12
ROWS = NP // NUM_SUBCORES
ZROWS = 40


def _dotT(a, w):
    return lax.dot_general(a, w, (((1,), (1,)), ((), ())),
                           preferred_element_type=jnp.float32)


def _swish(v):
    return v * jax.nn.sigmoid(v)


def _x1_body(x_ref, w_ref, b_ref, o_ref):
    o_ref[...] = _swish(_dotT(x_ref[...], w_ref[...]) + b_ref[...])


def _rnd_bf16(v):
    u = lax.bitcast_convert_type(v, jnp.uint32)
    return ((u + 0x7FFF + ((u >> 16) & 1)) >> 16).astype(jnp.uint32)


def _pack_bf16(y):
    w = _rnd_bf16(y[:, :H // 2]) | (_rnd_bf16(y[:, H // 2:]) << 16)
    return lax.bitcast_convert_type(w, jnp.int32)


def _dotTb(a, w):
    return lax.dot_general(a.astype(jnp.bfloat16), w.astype(jnp.bfloat16),
                           (((1,), (1,)), ((), ())),
                           preferred_element_type=jnp.float32)


def _edge_body(feat1_ref, feat2_ref, w11_ref, w12_ref, w21_ref, w22_ref,
               f1_ref, f2_ref):
    f1_ref[...] = _pack_bf16(_dotTb(_dotTb(feat1_ref[...], w11_ref[...]),
                                    w12_ref[...]))
    f2_ref[...] = _pack_bf16(_dotTb(_dotTb(feat2_ref[...], w21_ref[...]),
                                    w22_ref[...]))


EBLK = 4000


def _edge_features(feature1, feature2, f1_W1, f1_W2, f2_W1, f2_W2):
    grid = (E // EBLK,)
    return pl.pallas_call(
        _edge_body,
        grid=grid,
        in_specs=[
            pl.BlockSpec((EBLK, F1), lambda i: (i, 0)),
            pl.BlockSpec((EBLK, F2), lambda i: (i, 0)),
            pl.BlockSpec((MID, F1), lambda i: (0, 0)),
            pl.BlockSpec((H, MID), lambda i: (0, 0)),
            pl.BlockSpec((MID, F2), lambda i: (0, 0)),
            pl.BlockSpec((H, MID), lambda i: (0, 0)),
        ],
        out_specs=[
            pl.BlockSpec((EBLK, H // 2), lambda i: (i, 0)),
            pl.BlockSpec((EBLK, H // 2), lambda i: (i, 0)),
        ],
        out_shape=[
            jax.ShapeDtypeStruct((E, H // 2), jnp.int32),
            jax.ShapeDtypeStruct((E, H // 2), jnp.int32),
        ],
    )(feature1, feature2, f1_W1, f1_W2, f2_W1, f2_W2)


def _sc_body(x1_hbm, f1_hbm, f2_hbm, src_hbm, dst_hbm,
             out1_hbm, out2_hbm,
             agg_sp,
             srcv, dstv, fv, xg, mv, zbuf,
             sl, sg):
    c = lax.axis_index("c")
    s = lax.axis_index("s")
    wid = s * NUM_CORES + c
    row0 = s * ROWS

    def _zb(i, _):
        r = i // (H // 16)
        l = (i % (H // 16)) * 16
        zbuf[r, pl.ds(l, 16)] = jnp.zeros((16,), jnp.float32)
        return 0
    lax.fori_loop(0, ZROWS * (H // 16), _zb, 0)

    for f_hbm, out_hbm in ((f1_hbm, out1_hbm), (f2_hbm, out2_hbm)):
        for j in range(ROWS // ZROWS):
            pltpu.sync_copy(zbuf, agg_sp.at[pl.ds(row0 + j * ZROWS, ZROWS)])
        plsc.subcore_barrier()

        def load_start(k, b):
            e0 = wid * EW + k * CHUNK
            pltpu.async_copy(src_hbm.at[pl.ds(e0, CHUNK)], srcv[b], sl[b])
            pltpu.async_copy(dst_hbm.at[pl.ds(e0, CHUNK)], dstv[b], sl[b])
            pltpu.async_copy(f_hbm.at[pl.ds(e0, CHUNK)], fv[b], sl[b])

        def load_wait(b):
            pltpu.make_async_copy(src_hbm.at[pl.ds(0, CHUNK)], srcv[b], sl[b]).wait()
            pltpu.make_async_copy(dst_hbm.at[pl.ds(0, CHUNK)], dstv[b], sl[b]).wait()
            pltpu.make_async_copy(f_hbm.at[pl.ds(0, CHUNK)], fv[b], sl[b]).wait()

        def gather_start(lb, gb):
            pltpu.async_copy(x1_hbm.at[srcv[lb]], xg[gb], sg[gb])

        def gather_wait(gb):
            pltpu.make_async_copy(x1_hbm.at[pl.ds(0, CHUNK)], xg[gb], sg[gb]).wait()

        load_start(0, 0)
        load_start(1, 1)

        def _grp(jj, _):
            for p in range(UNROLL):
                j = UNROLL * jj + p
                lb = p % NBUF
                gb = p % NBG
                plb = (p + 2) % NBUF
                pgb = (p + 1) % NBG

                @pl.when(j < NCHUNK)
                def _():
                    load_wait(lb)
                    gather_start(lb, gb)

                @pl.when(j >= 2)
                def _():
                    gather_wait(pgb)

                    def _mul(r, _):
                        for g in range(4):
                            w = fv[plb][r, pl.ds(g * 16, 16)]
                            lo = lax.bitcast_convert_type(w << 16, jnp.float32)
                            hi = lax.bitcast_convert_type(
                                w & jnp.int32(-65536), jnp.float32)
                            mv[r, pl.ds(g * 16, 16)] = (
                                xg[pgb][r, pl.ds(g * 16, 16)] * lo)
                            mv[r, pl.ds(64 + g * 16, 16)] = (
                                xg[pgb][r, pl.ds(64 + g * 16, 16)] * hi)
                        return 0
                    lax.fori_loop(0, CHUNK, _mul, 0)

                    pltpu.sync_copy(mv, agg_sp.at[dstv[plb]], add=True)

                @pl.when(j + 2 < NCHUNK)
                def _():
                    load_start(j + 2, plb)
            return 0

        lax.fori_loop(0, (NCHUNK + 2) // UNROLL, _grp, 0)
        plsc.subcore_barrier()

        pltpu.sync_copy(agg_sp.at[pl.ds(row0, ROWS)],
                        out_hbm.at[c, pl.ds(row0, ROWS)])


def _sc_aggregate(x1, f1, f2, src, dst):
    mesh = plsc.VectorSubcoreMesh(core_axis_name="c", subcore_axis_name="s")
    fn = pl.kernel(
        _sc_body,
        out_type=[
            jax.ShapeDtypeStruct((NUM_CORES, NP, H), jnp.float32),
            jax.ShapeDtypeStruct((NUM_CORES, NP, H), jnp.float32),
        ],
        mesh=mesh,
        scratch_types=[
            pltpu.VMEM_SHARED((NP, H), jnp.float32),
            tuple(pltpu.VMEM((CHUNK,), jnp.int32) for _ in range(NBUF)),
            tuple(pltpu.VMEM((CHUNK,), jnp.int32) for _ in range(NBUF)),
            tuple(pltpu.VMEM((CHUNK, H // 2), jnp.int32) for _ in range(NBUF)),
            tuple(pltpu.VMEM((CHUNK, H), jnp.float32) for _ in range(NBG)),
            pltpu.VMEM((CHUNK, H), jnp.float32),
            pltpu.VMEM((ZROWS, H), jnp.float32),
            tuple(pltpu.SemaphoreType.DMA for _ in range(NBUF)),
            tuple(pltpu.SemaphoreType.DMA for _ in range(NBG)),
        ],
    )
    return fn(x1, f1, f2, src, dst)


NBLK = 1024


def _node1_body(parts1_ref, parts2_ref, x1_ref,
                c1_Wrel_ref, c1_brel_ref, c1_Wroot_ref,
                c2_Wrel_ref, c2_brel_ref, c2_Wroot_ref,
                lin1_W_ref, lin1_b_ref, lin2_W_ref, lin2_b_ref,
                cat_W_ref, cat_b_ref, lins_W_ref, lins_b_ref,
                h_ref):
    x1 = x1_ref[...]
    parts1 = parts1_ref[...]
    parts2 = parts2_ref[...]
    agg1 = parts1[0] + parts1[1]
    agg2 = parts2[0] + parts2[1]

    conv1 = _dotT(agg1, c1_Wrel_ref[...]) + c1_brel_ref[...] \
        + _dotT(x1, c1_Wroot_ref[...])
    conv2 = _dotT(agg2, c2_Wrel_ref[...]) + c2_brel_ref[...] \
        + _dotT(x1, c2_Wroot_ref[...])
    h1 = _swish(_dotT(conv1, lin1_W_ref[...]) + lin1_b_ref[...])
    h2 = _swish(_dotT(conv2, lin2_W_ref[...]) + lin2_b_ref[...])

    cat_W = cat_W_ref[...]
    h = _dotT(h1, cat_W[:, :H]) + _dotT(h2, cat_W[:, H:]) + cat_b_ref[...] + x1

    lins_W = lins_W_ref[...]
    lins_b = lins_b_ref[...]
    for i in range(NL):
        h = _swish(_dotT(h, lins_W[i]) + lins_b[i][None, :]) + h
    h_ref[...] = h


def _node2_body(h_ref, batch_ref, gn_w_ref, gn_b_ref, gn_ms_ref,
                final_W_ref, final_b_ref, out_ref):
    h = h_ref[...][:N]
    gids = jax.lax.broadcasted_iota(jnp.int32, (N, NG), 1)
    oh = (batch_ref[...] == gids).astype(jnp.float32)
    cnt = jnp.maximum(jnp.sum(oh, axis=0), 1.0)
    sums = lax.dot_general(oh, h, (((0,), (0,)), ((), ())),
                           preferred_element_type=jnp.float32)
    mean = sums / cnt[:, None]
    mean_n = lax.dot_general(oh, mean, (((1,), (0,)), ((), ())),
                             preferred_element_type=jnp.float32)
    outh = h - mean_n * gn_ms_ref[...]
    var = lax.dot_general(oh, outh * outh, (((0,), (0,)), ((), ())),
                          preferred_element_type=jnp.float32) / cnt[:, None]
    std = jnp.sqrt(var + 1e-5)
    std_n = lax.dot_general(oh, std, (((1,), (0,)), ((), ())),
                            preferred_element_type=jnp.float32)
    hn = gn_w_ref[...] * outh / std_n + gn_b_ref[...]
    out_ref[...] = _dotT(hn, final_W_ref[...]) + final_b_ref[...]


def _node_pipeline(parts1, parts2, x1, batch2d, w1, w2):
    h = pl.pallas_call(
        _node1_body,
        grid=(NP // NBLK,),
        in_specs=[
            pl.BlockSpec((NUM_CORES, NBLK, H), lambda i: (0, i, 0)),
            pl.BlockSpec((NUM_CORES, NBLK, H), lambda i: (0, i, 0)),
            pl.BlockSpec((NBLK, H), lambda i: (i, 0)),
        ] + [pl.BlockSpec(w.shape, lambda i, n=len(w.shape): (0,) * n)
             for w in w1],
        out_specs=pl.BlockSpec((NBLK, H), lambda i: (i, 0)),
        out_shape=jax.ShapeDtypeStruct((NP, H), jnp.float32),
    )(parts1, parts2, x1, *w1)
    return pl.pallas_call(
        _node2_body,
        out_shape=jax.ShapeDtypeStruct((N, H), jnp.float32),
    )(h, batch2d, *w2)


def kernel(x, feature1, feature2, edge_index, batch, lin_W, lin_b, f1_W1,
           f1_W2, f2_W1, f2_W2, c1_Wrel, c1_brel, c1_Wroot, c2_Wrel, c2_brel,
           c2_Wroot, lin1_W, lin1_b, lin2_W, lin2_b, cat_W, cat_b, lins_W,
           lins_b, gn_w, gn_b, gn_ms, final_W, final_b):
    src = edge_index[0]
    dst = edge_index[1]

    x_pad = jnp.pad(x, ((0, NP - N), (0, 0)))
    x1 = pl.pallas_call(
        _x1_body,
        out_shape=jax.ShapeDtypeStruct((NP, H), jnp.float32),
    )(x_pad, lin_W, lin_b.reshape(1, H))

    f1, f2 = _edge_features(feature1, feature2, f1_W1, f1_W2, f2_W1, f2_W2)

    parts1, parts2 = _sc_aggregate(x1, f1, f2, src, dst)

    w1 = (
        c1_Wrel, c1_brel.reshape(1, H), c1_Wroot,
        c2_Wrel, c2_brel.reshape(1, H), c2_Wroot,
        lin1_W, lin1_b.reshape(1, H), lin2_W, lin2_b.reshape(1, H),
        cat_W, cat_b.reshape(1, H), lins_W, lins_b,
    )
    w2 = (
        gn_w.reshape(1, H), gn_b.reshape(1, H), gn_ms.reshape(1, H),
        final_W, final_b.reshape(1, H),
    )
    return _node_pipeline(parts1, parts2, x1, batch.reshape(N, 1), w1, w2)


# speedup vs baseline: 2.7791x; 1.0053x over previous
import jax
import jax.numpy as jnp
from jax import lax
from jax.experimental import pallas as pl
from jax.experimental.pallas import tpu as pltpu
from jax.experimental.pallas import tpu_sc as plsc

N = 10000
NP = 10240
E = 320000
H = 128
MID = 64
F1 = 147
F2 = 21
NL = 4
NG = 64

NUM_CORES = 2
NUM_SUBCORES = 16
NW = NUM_CORES * NUM_SUBCORES
EW = E // NW
CHUNK = 40
NCHUNK = EW // CHUNK
NBUF = 4
NBG = 3
UNROLL = 12
ROWS = NP // NUM_SUBCORES
ZROWS = 40


def _dotT(a, w):
    return lax.dot_general(a, w, (((1,), (1,)), ((), ())),
                           preferred_element_type=jnp.float32)


def _swish(v):
    return v * jax.nn.sigmoid(v)


def _x1_body(x_ref, w_ref, b_ref, o_ref):
    o_ref[...] = _swish(_dotT(x_ref[...], w_ref[...]) + b_ref[...])


def _rnd_bf16(v):
    u = lax.bitcast_convert_type(v, jnp.uint32)
    return ((u + 0x7FFF + ((u >> 16) & 1)) >> 16).astype(jnp.uint32)


def _pack_bf16(y):
    w = _rnd_bf16(y[:, :H // 2]) | (_rnd_bf16(y[:, H // 2:]) << 16)
    return lax.bitcast_convert_type(w, jnp.int32)


def _dotTb(a, w):
    return lax.dot_general(a.astype(jnp.bfloat16), w.astype(jnp.bfloat16),
                           (((1,), (1,)), ((), ())),
                           preferred_element_type=jnp.float32)


def _edge_body(feat1_ref, feat2_ref, w11_ref, w12_ref, w21_ref, w22_ref,
               f1_ref, f2_ref):
    f1_ref[...] = _pack_bf16(_dotTb(_dotTb(feat1_ref[...], w11_ref[...]),
                                    w12_ref[...]))
    f2_ref[...] = _pack_bf16(_dotTb(_dotTb(feat2_ref[...], w21_ref[...]),
                                    w22_ref[...]))


EBLK = 4000


def _edge_features(feature1, feature2, f1_W1, f1_W2, f2_W1, f2_W2):
    grid = (E // EBLK,)
    return pl.pallas_call(
        _edge_body,
        grid=grid,
        in_specs=[
            pl.BlockSpec((EBLK, F1), lambda i: (i, 0)),
            pl.BlockSpec((EBLK, F2), lambda i: (i, 0)),
            pl.BlockSpec((MID, F1), lambda i: (0, 0)),
            pl.BlockSpec((H, MID), lambda i: (0, 0)),
            pl.BlockSpec((MID, F2), lambda i: (0, 0)),
            pl.BlockSpec((H, MID), lambda i: (0, 0)),
        ],
        out_specs=[
            pl.BlockSpec((EBLK, H // 2), lambda i: (i, 0)),
            pl.BlockSpec((EBLK, H // 2), lambda i: (i, 0)),
        ],
        out_shape=[
            jax.ShapeDtypeStruct((E, H // 2), jnp.int32),
            jax.ShapeDtypeStruct((E, H // 2), jnp.int32),
        ],
    )(feature1, feature2, f1_W1, f1_W2, f2_W1, f2_W2)


def _sc_body(x1_hbm, f1_hbm, f2_hbm, src_hbm, dst_hbm,
             out1_hbm, out2_hbm,
             agg_sp,
             srcv, dstv, fv, xg, mv, zbuf,
             sl, sg):
    c = lax.axis_index("c")
    s = lax.axis_index("s")
    wid = s * NUM_CORES + c
    row0 = s * ROWS

    def _zb(i, _):
        r = i // (H // 16)
        l = (i % (H // 16)) * 16
        zbuf[r, pl.ds(l, 16)] = jnp.zeros((16,), jnp.float32)
        return 0
    lax.fori_loop(0, ZROWS * (H // 16), _zb, 0)

    for f_hbm, out_hbm in ((f1_hbm, out1_hbm), (f2_hbm, out2_hbm)):
        for j in range(ROWS // ZROWS):
            pltpu.sync_copy(zbuf, agg_sp.at[pl.ds(row0 + j * ZROWS, ZROWS)])
        plsc.subcore_barrier()

        def load_start(k, b):
            e0 = wid * EW + k * CHUNK
            pltpu.async_copy(src_hbm.at[pl.ds(e0, CHUNK)], srcv[b], sl[b])
            pltpu.async_copy(dst_hbm.at[pl.ds(e0, CHUNK)], dstv[b], sl[b])
            pltpu.async_copy(f_hbm.at[pl.ds(e0, CHUNK)], fv[b], sl[b])

        def load_wait(b):
            pltpu.make_async_copy(src_hbm.at[pl.ds(0, CHUNK)], srcv[b], sl[b]).wait()
            pltpu.make_async_copy(dst_hbm.at[pl.ds(0, CHUNK)], dstv[b], sl[b]).wait()
            pltpu.make_async_copy(f_hbm.at[pl.ds(0, CHUNK)], fv[b], sl[b]).wait()

        def gather_start(lb, gb):
            pltpu.async_copy(x1_hbm.at[srcv[lb]], xg[gb], sg[gb])

        def gather_wait(gb):
            pltpu.make_async_copy(x1_hbm.at[pl.ds(0, CHUNK)], xg[gb], sg[gb]).wait()

        load_start(0, 0)
        load_start(1, 1)

        def _grp(jj, _):
            for p in range(UNROLL):
                j = UNROLL * jj + p
                lb = p % NBUF
                gb = p % NBG
                plb = (p + 2) % NBUF
                pgb = (p + 1) % NBG

                @pl.when(j < NCHUNK)
                def _():
                    load_wait(lb)
                    gather_start(lb, gb)

                @pl.when(j >= 2)
                def _():
                    gather_wait(pgb)

                    def _mul(r, _):
                        for g in range(4):
                            w = fv[plb][r, pl.ds(g * 16, 16)]
                            lo = lax.bitcast_convert_type(w << 16, jnp.float32)
                            hi = lax.bitcast_convert_type(
                                w & jnp.int32(-65536), jnp.float32)
                            mv[r, pl.ds(g * 16, 16)] = (
                                xg[pgb][r, pl.ds(g * 16, 16)] * lo)
                            mv[r, pl.ds(64 + g * 16, 16)] = (
                                xg[pgb][r, pl.ds(64 + g * 16, 16)] * hi)
                        return 0
                    lax.fori_loop(0, CHUNK, _mul, 0)

                    pltpu.sync_copy(mv, agg_sp.at[dstv[plb]], add=True)

                @pl.when(j + 2 < NCHUNK)
                def _():
                    load_start(j + 2, plb)
            return 0

        lax.fori_loop(0, (NCHUNK + 2) // UNROLL, _grp, 0)
        plsc.subcore_barrier()

        pltpu.sync_copy(agg_sp.at[pl.ds(row0, ROWS)],
                        out_hbm.at[c, pl.ds(row0, ROWS)])


def _sc_aggregate(x1, f1, f2, src, dst):
    mesh = plsc.VectorSubcoreMesh(core_axis_name="c", subcore_axis_name="s")
    fn = pl.kernel(
        _sc_body,
        out_type=[
            jax.ShapeDtypeStruct((NUM_CORES, NP, H), jnp.float32),
            jax.ShapeDtypeStruct((NUM_CORES, NP, H), jnp.float32),
        ],
        mesh=mesh,
        scratch_types=[
            pltpu.VMEM_SHARED((NP, H), jnp.float32),
            tuple(pltpu.VMEM((CHUNK,), jnp.int32) for _ in range(NBUF)),
            tuple(pltpu.VMEM((CHUNK,), jnp.int32) for _ in range(NBUF)),
            tuple(pltpu.VMEM((CHUNK, H // 2), jnp.int32) for _ in range(NBUF)),
            tuple(pltpu.VMEM((CHUNK, H), jnp.float32) for _ in range(NBG)),
            pltpu.VMEM((CHUNK, H), jnp.float32),
            pltpu.VMEM((ZROWS, H), jnp.float32),
            tuple(pltpu.SemaphoreType.DMA for _ in range(NBUF)),
            tuple(pltpu.SemaphoreType.DMA for _ in range(NBG)),
        ],
    )
    return fn(x1, f1, f2, src, dst)


NBLK = 2000


def _node1_body(parts1_ref, parts2_ref, x1_ref,
                c1_Wrel_ref, c1_brel_ref, c1_Wroot_ref,
                c2_Wrel_ref, c2_brel_ref, c2_Wroot_ref,
                lin1_W_ref, lin1_b_ref, lin2_W_ref, lin2_b_ref,
                cat_W_ref, cat_b_ref, lins_W_ref, lins_b_ref,
                h_ref):
    x1 = x1_ref[...]
    parts1 = parts1_ref[...]
    parts2 = parts2_ref[...]
    agg1 = parts1[0] + parts1[1]
    agg2 = parts2[0] + parts2[1]

    conv1 = _dotT(agg1, c1_Wrel_ref[...]) + c1_brel_ref[...] \
        + _dotT(x1, c1_Wroot_ref[...])
    conv2 = _dotT(agg2, c2_Wrel_ref[...]) + c2_brel_ref[...] \
        + _dotT(x1, c2_Wroot_ref[...])
    h1 = _swish(_dotT(conv1, lin1_W_ref[...]) + lin1_b_ref[...])
    h2 = _swish(_dotT(conv2, lin2_W_ref[...]) + lin2_b_ref[...])

    cat_W = cat_W_ref[...]
    h = _dotT(h1, cat_W[:, :H]) + _dotT(h2, cat_W[:, H:]) + cat_b_ref[...] + x1

    lins_W = lins_W_ref[...]
    lins_b = lins_b_ref[...]
    for i in range(NL):
        h = _swish(_dotT(h, lins_W[i]) + lins_b[i][None, :]) + h
    h_ref[...] = h


def _node2_body(h_ref, batch_ref, gn_w_ref, gn_b_ref, gn_ms_ref,
                final_W_ref, final_b_ref, out_ref):
    h = h_ref[...]
    gids = jax.lax.broadcasted_iota(jnp.int32, (N, NG), 1)
    oh = (batch_ref[...] == gids).astype(jnp.float32)
    cnt = jnp.maximum(jnp.sum(oh, axis=0), 1.0)
    sums = lax.dot_general(oh, h, (((0,), (0,)), ((), ())),
                           preferred_element_type=jnp.float32)
    mean = sums / cnt[:, None]
    mean_n = lax.dot_general(oh, mean, (((1,), (0,)), ((), ())),
                             preferred_element_type=jnp.float32)
    outh = h - mean_n * gn_ms_ref[...]
    var = lax.dot_general(oh, outh * outh, (((0,), (0,)), ((), ())),
                          preferred_element_type=jnp.float32) / cnt[:, None]
    std = jnp.sqrt(var + 1e-5)
    std_n = lax.dot_general(oh, std, (((1,), (0,)), ((), ())),
                            preferred_element_type=jnp.float32)
    hn = gn_w_ref[...] * outh / std_n + gn_b_ref[...]
    out_ref[...] = _dotT(hn, final_W_ref[...]) + final_b_ref[...]


def _node_pipeline(parts1, parts2, x1, batch2d, w1, w2):
    h = pl.pallas_call(
        _node1_body,
        grid=(N // NBLK,),
        in_specs=[
            pl.BlockSpec((NUM_CORES, NBLK, H), lambda i: (0, i, 0)),
            pl.BlockSpec((NUM_CORES, NBLK, H), lambda i: (0, i, 0)),
            pl.BlockSpec((NBLK, H), lambda i: (i, 0)),
        ] + [pl.BlockSpec(w.shape, lambda i, n=len(w.shape): (0,) * n)
             for w in w1],
        out_specs=pl.BlockSpec((NBLK, H), lambda i: (i, 0)),
        out_shape=jax.ShapeDtypeStruct((N, H), jnp.float32),
    )(parts1, parts2, x1, *w1)
    return pl.pallas_call(
        _node2_body,
        out_shape=jax.ShapeDtypeStruct((N, H), jnp.float32),
    )(h, batch2d, *w2)


def kernel(x, feature1, feature2, edge_index, batch, lin_W, lin_b, f1_W1,
           f1_W2, f2_W1, f2_W2, c1_Wrel, c1_brel, c1_Wroot, c2_Wrel, c2_brel,
           c2_Wroot, lin1_W, lin1_b, lin2_W, lin2_b, cat_W, cat_b, lins_W,
           lins_b, gn_w, gn_b, gn_ms, final_W, final_b):
    src = edge_index[0]
    dst = edge_index[1]

    x1 = pl.pallas_call(
        _x1_body,
        out_shape=jax.ShapeDtypeStruct((N, H), jnp.float32),
    )(x, lin_W, lin_b.reshape(1, H))

    f1, f2 = _edge_features(feature1, feature2, f1_W1, f1_W2, f2_W1, f2_W2)

    parts1, parts2 = _sc_aggregate(x1, f1, f2, src, dst)

    w1 = (
        c1_Wrel, c1_brel.reshape(1, H), c1_Wroot,
        c2_Wrel, c2_brel.reshape(1, H), c2_Wroot,
        lin1_W, lin1_b.reshape(1, H), lin2_W, lin2_b.reshape(1, H),
        cat_W, cat_b.reshape(1, H), lins_W, lins_b,
    )
    w2 = (
        gn_w.reshape(1, H), gn_b.reshape(1, H), gn_ms.reshape(1, H),
        final_W, final_b.reshape(1, H),
    )
    return _node_pipeline(parts1, parts2, x1, batch.reshape(N, 1), w1, w2)
